# baseline (speedup 1.0000x reference)
import functools

import jax
import jax.numpy as jnp
from jax.experimental import pallas as pl
from jax.experimental.pallas import tpu as pltpu

B, T, L, F = 16, 32, 4, 1024
OUT_POINTS = 4096
DIM = 3
M = B * T
K = L * F
N = OUT_POINTS * DIM

BK = 256
SPF = F // BK
H = N // 2


def _matmul_body(x_ref, w_ref, b_ref, o_ref):
    k = pl.program_id(0)

    @pl.when(k == 0)
    def _():
        o_ref[...] = jnp.broadcast_to(b_ref[...], o_ref.shape)

    o_ref[...] += jnp.dot(x_ref[...].astype(jnp.bfloat16),
                          w_ref[...].astype(jnp.bfloat16),
                          preferred_element_type=jnp.float32)


def _x_index(k):
    l = k // SPF
    return (0, (L - 1 - l) * SPF + (k % SPF))


def _half(x, W, b2, co):
    return pl.pallas_call(
        _matmul_body,
        grid=(K // BK,),
        in_specs=[
            pl.BlockSpec((M, BK), _x_index),
            pl.BlockSpec((BK, H), lambda k: (k, co)),
            pl.BlockSpec((1, H), lambda k: (0, co)),
        ],
        out_specs=pl.BlockSpec((M, H), lambda k: (0, 0)),
        out_shape=jax.ShapeDtypeStruct((M, H), jnp.float32),
        compiler_params=pltpu.CompilerParams(
            dimension_semantics=("arbitrary",),
        ),
    )(x, W, b2)


@jax.jit
def _decode(rnn, W, b):
    x = rnn.reshape(M, K)
    b2 = b.reshape(1, N)
    halves = [
        _half(x, W, b2, co).reshape(B, T, OUT_POINTS // 2, DIM)
        for co in range(2)
    ]
    return jnp.concatenate(halves, axis=2)


def kernel(rnn, enc_xyz, enc_feats, W, b):
    del enc_xyz, enc_feats
    return _decode(rnn, W, b)


# speedup vs baseline: 1.1202x; 1.1202x over previous
import jax
import jax.numpy as jnp
from jax.experimental import pallas as pl
from jax.experimental.pallas import tpu as pltpu

B, T, L, F = 16, 32, 4, 1024
OUT_POINTS = 4096
DIM = 3
M = B * T
K = L * F
N = OUT_POINTS * DIM

BK = 256
SPF = F // BK


def _matmul_body(x_ref, w_ref, b_ref, o_ref):
    k = pl.program_id(0)

    @pl.when(k == 0)
    def _():
        o_ref[...] = jnp.broadcast_to(b_ref[...], o_ref.shape)

    l = k // SPF
    f0 = (k % SPF) * BK
    xs = x_ref[:, :, L - 1 - l, pl.ds(f0, BK)]
    xs = xs.reshape(M, BK)
    o_ref[...] += jnp.dot(xs.astype(jnp.bfloat16),
                          w_ref[...].astype(jnp.bfloat16),
                          preferred_element_type=jnp.float32)


@jax.jit
def _decode(rnn, W, b):
    b2 = b.reshape(1, N)
    out = pl.pallas_call(
        _matmul_body,
        grid=(K // BK,),
        in_specs=[
            pl.BlockSpec((B, T, L, F), lambda k: (0, 0, 0, 0)),
            pl.BlockSpec((BK, N), lambda k: (k, 0)),
            pl.BlockSpec((1, N), lambda k: (0, 0)),
        ],
        out_specs=pl.BlockSpec((M, N), lambda k: (0, 0)),
        out_shape=jax.ShapeDtypeStruct((M, N), jnp.float32),
        compiler_params=pltpu.CompilerParams(
            dimension_semantics=("arbitrary",),
        ),
    )(rnn, W, b2)
    return out.reshape(B, T, OUT_POINTS, DIM)


def kernel(rnn, enc_xyz, enc_feats, W, b):
    del enc_xyz, enc_feats
    return _decode(rnn, W, b)
